# baseline (speedup 1.0000x reference)
import jax
import jax.numpy as jnp
from jax import lax
from jax.experimental import pallas as pl
from jax.experimental.pallas import tpu as pltpu

_N = 100000
_D = 128
_B = 2000
_G = _B // 8


def _body(keys_ref, q_ref, out_ref, vmin_ref, gwin_ref):
    pid = pl.program_id(0)

    @pl.when(pid == 0)
    def _init():
        vmin_ref[...] = jnp.full((8, _D), jnp.inf, jnp.float32)
        gwin_ref[...] = jnp.zeros((8, _D), jnp.int32)

    x = jnp.abs(keys_ref[...] - q_ref[...])
    for shift in (64, 32, 16, 8, 4, 2, 1):
        x = x + pltpu.roll(x, shift, 1)
    y = x.reshape(_G, 8, _D)
    mval = jnp.min(y, axis=0)
    g3 = lax.broadcasted_iota(jnp.int32, (_G, 8, _D), 0)
    gwin = jnp.min(jnp.where(y == mval[None], g3, jnp.int32(_G)), axis=0)

    upd = mval < vmin_ref[...]
    vmin_ref[...] = jnp.where(upd, mval, vmin_ref[...])
    gwin_ref[...] = jnp.where(upd, pid * _G + gwin, gwin_ref[...])

    @pl.when(pid == pl.num_programs(0) - 1)
    def _emit():
        acc = vmin_ref[...]
        m = jnp.min(acc)
        sub = lax.broadcasted_iota(jnp.int32, (8, _D), 0)
        ridx = gwin_ref[...] * 8 + sub
        out_ref[0] = jnp.min(jnp.where(acc == m, ridx, jnp.int32(_N)))


def kernel(keys, query):
    out = pl.pallas_call(
        _body,
        grid=(_N // _B,),
        in_specs=[
            pl.BlockSpec((_B, _D), lambda i: (i, 0)),
            pl.BlockSpec((1, _D), lambda i: (0, 0)),
        ],
        out_specs=pl.BlockSpec(memory_space=pltpu.SMEM),
        out_shape=jax.ShapeDtypeStruct((1,), jnp.int32),
        scratch_shapes=[
            pltpu.VMEM((8, _D), jnp.float32),
            pltpu.VMEM((8, _D), jnp.int32),
        ],
    )(keys, query)
    return out[0]
